# baseline (speedup 1.0000x reference)
import functools

import jax
import jax.numpy as jnp
from jax import lax
from jax.experimental import pallas as pl
from jax.experimental.pallas import tpu as pltpu
from jax.experimental.pallas import tpu_sc as plsc


_NC = 2
_NS = 16
_NW = _NC * _NS


def _sc_copy_body(data_hbm, out_hbm, idx_v, buf_v, gsem, wsem):
    c = lax.axis_index("c")
    s = lax.axis_index("s")
    wid = s * _NC + c
    n = data_hbm.shape[0]
    rows_per_w = n // _NW
    base = wid * rows_per_w

    for j in range(rows_per_w // 16):
        idx_v[pl.ds(j * 16, 16)] = base + j * 16 + lax.iota(jnp.int32, 16)

    rpr = 32
    rounds = rows_per_w // rpr

    def gather(r, slot):
        return pltpu.async_copy(
            data_hbm.at[idx_v.at[pl.ds(r * rpr, rpr)]], buf_v.at[slot], gsem)

    def write(r, slot):
        return pltpu.async_copy(
            buf_v.at[slot], out_hbm.at[pl.ds(base + r * rpr, rpr)], wsem)

    g = {0: gather(0, 0)}
    w = {}
    for r in range(rounds):
        g[r].wait()
        if r >= 1:
            w[r - 1].wait()
        if r < rounds - 1:
            g[r + 1] = gather(r + 1, (r + 1) % 2)
        w[r] = write(r, r % 2)
    w[rounds - 1].wait()


def _sc_copy(data):
    n, d = data.shape
    mesh = plsc.VectorSubcoreMesh(core_axis_name="c", subcore_axis_name="s")
    return pl.kernel(
        _sc_copy_body,
        out_type=jax.ShapeDtypeStruct((n, d), data.dtype),
        mesh=mesh,
        scratch_types=[
            pltpu.VMEM((n // _NW,), jnp.int32),
            pltpu.VMEM((2, 32, d), data.dtype),
            pltpu.SemaphoreType.DMA,
            pltpu.SemaphoreType.DMA,
        ],
    )(data)


def _mm_body(mask_ref, a_ref, w_ref, b_ref, o_ref):
    acc = jax.lax.dot_general(
        a_ref[...], w_ref[...], (((1,), (1,)), ((), ())),
        preferred_element_type=jnp.float32)
    o_ref[...] = (acc + b_ref[...]) * mask_ref[...]


def kernel(indicators, data, W, b):
    N, d_in = data.shape
    d_out = W.shape[0]
    BM = 512
    maskf = (indicators != 0).astype(jnp.float32).reshape(N, 1)
    datac = _sc_copy(data)
    out = pl.pallas_call(
        _mm_body,
        grid=(N // BM,),
        in_specs=[
            pl.BlockSpec((BM, 1), lambda i: (i, 0)),
            pl.BlockSpec((BM, d_in), lambda i: (i, 0)),
            pl.BlockSpec((d_out, d_in), lambda i: (0, 0)),
            pl.BlockSpec((1, d_out), lambda i: (0, 0)),
        ],
        out_specs=pl.BlockSpec((BM, d_out), lambda i: (i, 0)),
        out_shape=jax.ShapeDtypeStruct((N, d_out), jnp.float32),
    )(maskf, datac, W, b.reshape(1, d_out))
    return out


# speedup vs baseline: 1.6363x; 1.6363x over previous
import functools

import jax
import jax.numpy as jnp
from jax import lax
from jax.experimental import pallas as pl
from jax.experimental.pallas import tpu as pltpu
from jax.experimental.pallas import tpu_sc as plsc


_NC = 2
_NS = 16
_NW = _NC * _NS


def _sc_copy_body(data_hbm, out_hbm, idx_v, buf_v, gsem, wsem):
    c = lax.axis_index("c")
    s = lax.axis_index("s")
    wid = s * _NC + c
    n = data_hbm.shape[0]
    rows_per_w = n // _NW
    base = wid * rows_per_w

    for j in range(rows_per_w // 16):
        idx_v[pl.ds(j * 16, 16)] = base + j * 16 + lax.iota(jnp.int32, 16)

    rpr = 32
    rounds = rows_per_w // rpr

    def gather(r, slot):
        return pltpu.async_copy(
            data_hbm.at[idx_v.at[pl.ds(r * rpr, rpr)]], buf_v.at[slot], gsem)

    def write(r, slot):
        return pltpu.async_copy(
            buf_v.at[slot], out_hbm.at[pl.ds(base + r * rpr, rpr)], wsem)

    g = {0: gather(0, 0)}
    w = {}
    for r in range(rounds):
        g[r].wait()
        if r >= 1:
            w[r - 1].wait()
        if r < rounds - 1:
            g[r + 1] = gather(r + 1, (r + 1) % 2)
        w[r] = write(r, r % 2)
    w[rounds - 1].wait()


def _sc_copy(data):
    n, d = data.shape
    mesh = plsc.VectorSubcoreMesh(core_axis_name="c", subcore_axis_name="s")
    return pl.kernel(
        _sc_copy_body,
        out_type=jax.ShapeDtypeStruct((n, d), data.dtype),
        mesh=mesh,
        scratch_types=[
            pltpu.VMEM((n // _NW,), jnp.int32),
            pltpu.VMEM((2, 32, d), data.dtype),
            pltpu.SemaphoreType.DMA,
            pltpu.SemaphoreType.DMA,
        ],
    )(data)


def _mm_body(mask_ref, a_ref, w_ref, b_ref, o_ref):
    acc = jax.lax.dot_general(
        a_ref[...], w_ref[...], (((1,), (1,)), ((), ())),
        preferred_element_type=jnp.float32)
    o_ref[...] = (acc + b_ref[...]) * mask_ref[...]


def kernel(indicators, data, W, b):
    N, d_in = data.shape
    d_out = W.shape[0]
    BM = 256
    maskf = (indicators != 0).astype(jnp.float32).reshape(N, 1)
    datac = data
    out = pl.pallas_call(
        _mm_body,
        grid=(N // BM,),
        in_specs=[
            pl.BlockSpec((BM, 1), lambda i: (i, 0)),
            pl.BlockSpec((BM, d_in), lambda i: (i, 0)),
            pl.BlockSpec((d_out, d_in), lambda i: (0, 0)),
            pl.BlockSpec((1, d_out), lambda i: (0, 0)),
        ],
        out_specs=pl.BlockSpec((BM, d_out), lambda i: (i, 0)),
        out_shape=jax.ShapeDtypeStruct((N, d_out), jnp.float32),
    )(maskf, datac, W, b.reshape(1, d_out))
    return out


# speedup vs baseline: 2.3731x; 1.4504x over previous
import functools

import jax
import jax.numpy as jnp
from jax import lax
from jax.experimental import pallas as pl
from jax.experimental.pallas import tpu as pltpu
from jax.experimental.pallas import tpu_sc as plsc


_NC = 2
_NS = 16
_NW = _NC * _NS


def _sc_copy_body(data_hbm, out_hbm, idx_v, buf_v, gsem, wsem):
    c = lax.axis_index("c")
    s = lax.axis_index("s")
    wid = s * _NC + c
    n = data_hbm.shape[0]
    rows_per_w = n // _NW
    base = wid * rows_per_w

    for j in range(rows_per_w // 16):
        idx_v[pl.ds(j * 16, 16)] = base + j * 16 + lax.iota(jnp.int32, 16)

    rpr = 32
    rounds = rows_per_w // rpr

    def gather(r, slot):
        return pltpu.async_copy(
            data_hbm.at[idx_v.at[pl.ds(r * rpr, rpr)]], buf_v.at[slot], gsem)

    def write(r, slot):
        return pltpu.async_copy(
            buf_v.at[slot], out_hbm.at[pl.ds(base + r * rpr, rpr)], wsem)

    g = {0: gather(0, 0)}
    w = {}
    for r in range(rounds):
        g[r].wait()
        if r >= 1:
            w[r - 1].wait()
        if r < rounds - 1:
            g[r + 1] = gather(r + 1, (r + 1) % 2)
        w[r] = write(r, r % 2)
    w[rounds - 1].wait()


def _sc_copy(data):
    n, d = data.shape
    mesh = plsc.VectorSubcoreMesh(core_axis_name="c", subcore_axis_name="s")
    return pl.kernel(
        _sc_copy_body,
        out_type=jax.ShapeDtypeStruct((n, d), data.dtype),
        mesh=mesh,
        scratch_types=[
            pltpu.VMEM((n // _NW,), jnp.int32),
            pltpu.VMEM((2, 32, d), data.dtype),
            pltpu.SemaphoreType.DMA,
            pltpu.SemaphoreType.DMA,
        ],
    )(data)


def _mm_body(mask_ref, a_ref, w_ref, b_ref, o_ref):
    acc = jax.lax.dot_general(
        a_ref[...], w_ref[...], (((1,), (1,)), ((), ())),
        preferred_element_type=jnp.float32)
    o_ref[...] = (acc + b_ref[...]) * mask_ref[...]


def kernel(indicators, data, W, b):
    N, d_in = data.shape
    d_out = W.shape[0]
    BM = 1024
    maskf = (indicators != 0).astype(jnp.float32).reshape(N, 1)
    datac = data
    out = pl.pallas_call(
        _mm_body,
        grid=(N // BM,),
        in_specs=[
            pl.BlockSpec((BM, 1), lambda i: (i, 0)),
            pl.BlockSpec((BM, d_in), lambda i: (i, 0)),
            pl.BlockSpec((d_out, d_in), lambda i: (0, 0)),
            pl.BlockSpec((1, d_out), lambda i: (0, 0)),
        ],
        out_specs=pl.BlockSpec((BM, d_out), lambda i: (i, 0)),
        out_shape=jax.ShapeDtypeStruct((N, d_out), jnp.float32),
    )(maskf, datac, W, b.reshape(1, d_out))
    return out


# speedup vs baseline: 2.3895x; 1.0069x over previous
import functools

import jax
import jax.numpy as jnp
from jax import lax
from jax.experimental import pallas as pl
from jax.experimental.pallas import tpu as pltpu
from jax.experimental.pallas import tpu_sc as plsc


_NC = 2
_NS = 16
_NW = _NC * _NS


def _sc_copy_body(data_hbm, out_hbm, idx_v, buf_v, gsem, wsem):
    c = lax.axis_index("c")
    s = lax.axis_index("s")
    wid = s * _NC + c
    n = data_hbm.shape[0]
    rows_per_w = n // _NW
    base = wid * rows_per_w

    for j in range(rows_per_w // 16):
        idx_v[pl.ds(j * 16, 16)] = base + j * 16 + lax.iota(jnp.int32, 16)

    rpr = 32
    rounds = rows_per_w // rpr

    def gather(r, slot):
        return pltpu.async_copy(
            data_hbm.at[idx_v.at[pl.ds(r * rpr, rpr)]], buf_v.at[slot], gsem)

    def write(r, slot):
        return pltpu.async_copy(
            buf_v.at[slot], out_hbm.at[pl.ds(base + r * rpr, rpr)], wsem)

    g = {0: gather(0, 0)}
    w = {}
    for r in range(rounds):
        g[r].wait()
        if r >= 1:
            w[r - 1].wait()
        if r < rounds - 1:
            g[r + 1] = gather(r + 1, (r + 1) % 2)
        w[r] = write(r, r % 2)
    w[rounds - 1].wait()


def _sc_copy(data):
    n, d = data.shape
    mesh = plsc.VectorSubcoreMesh(core_axis_name="c", subcore_axis_name="s")
    return pl.kernel(
        _sc_copy_body,
        out_type=jax.ShapeDtypeStruct((n, d), data.dtype),
        mesh=mesh,
        scratch_types=[
            pltpu.VMEM((n // _NW,), jnp.int32),
            pltpu.VMEM((2, 32, d), data.dtype),
            pltpu.SemaphoreType.DMA,
            pltpu.SemaphoreType.DMA,
        ],
    )(data)


def _mm_body(mask_ref, a_ref, w_ref, b_ref, o_ref):
    acc = jax.lax.dot_general(
        a_ref[...], w_ref[...], (((1,), (1,)), ((), ())),
        preferred_element_type=jnp.float32)
    o_ref[...] = (acc + b_ref[...]) * mask_ref[...]


def kernel(indicators, data, W, b):
    N, d_in = data.shape
    d_out = W.shape[0]
    BM = 2048
    maskf = (indicators != 0).astype(jnp.float32).reshape(N, 1)
    datac = data
    out = pl.pallas_call(
        _mm_body,
        grid=(N // BM,),
        in_specs=[
            pl.BlockSpec((BM, 1), lambda i: (i, 0)),
            pl.BlockSpec((BM, d_in), lambda i: (i, 0)),
            pl.BlockSpec((d_out, d_in), lambda i: (0, 0)),
            pl.BlockSpec((1, d_out), lambda i: (0, 0)),
        ],
        out_specs=pl.BlockSpec((BM, d_out), lambda i: (i, 0)),
        out_shape=jax.ShapeDtypeStruct((N, d_out), jnp.float32),
    )(maskf, datac, W, b.reshape(1, d_out))
    return out
